# Initial kernel scaffold; baseline (speedup 1.0000x reference)
import jax
import jax.numpy as jnp
from jax.experimental import pallas as pl


def kernel(data, batch_id, weights, bias):
    raise NotImplementedError("write your pallas kernel here")


# speedup vs baseline: 4.3919x; 4.3919x over previous
import functools

import jax
import jax.numpy as jnp
import numpy as np
from jax.experimental import pallas as pl
from jax.experimental.pallas import tpu as pltpu

IC = 512
NGROUPS = 32
CPG = IC // NGROUPS
EPS = 1e-5
NB = 16

_PREC = jax.lax.Precision.HIGHEST


def _pick_rows(n):
    for r in (4000, 2000, 1000, 800, 400, 200, 80, 40, 8):
        if n % r == 0:
            return r
    return n


def _stats_kernel(ids_ref, x_ref, s1_ref, s2_ref, cnt_ref, *, rows):
    x = x_ref[...]
    ids = ids_ref[...]
    iota = jax.lax.broadcasted_iota(jnp.int32, (rows, NB), 1)
    onehot = (ids == iota).astype(jnp.float32)
    dims = (((0,), (0,)), ((), ()))
    s1 = jax.lax.dot_general(onehot, x, dims,
                             preferred_element_type=jnp.float32,
                             precision=_PREC)
    s2 = jax.lax.dot_general(onehot, x * x, dims,
                             preferred_element_type=jnp.float32,
                             precision=_PREC)
    ones = jnp.ones((rows, 128), jnp.float32)
    cnt = jax.lax.dot_general(onehot, ones, dims,
                              preferred_element_type=jnp.float32,
                              precision=_PREC)

    @pl.when(pl.program_id(0) == 0)
    def _init():
        s1_ref[...] = s1
        s2_ref[...] = s2
        cnt_ref[...] = cnt

    @pl.when(pl.program_id(0) != 0)
    def _acc():
        s1_ref[...] += s1
        s2_ref[...] += s2
        cnt_ref[...] += cnt


def _table_kernel(s1_ref, s2_ref, cnt_ref, g_ref, w_ref, b_ref,
                  scale_ref, shift_ref):
    n16 = cnt_ref[...][:, 0:1] * CPG
    ic = 1.0 / (n16 + EPS)
    dims = (((1,), (0,)), ((), ()))
    s1g = jax.lax.dot_general(s1_ref[...], g_ref[...], dims,
                              preferred_element_type=jnp.float32,
                              precision=_PREC)
    s2g = jax.lax.dot_general(s2_ref[...], g_ref[...], dims,
                              preferred_element_type=jnp.float32,
                              precision=_PREC)
    m = s1g * ic
    q = n16 * ic
    var = s2g * ic - m * m * (2.0 - q)
    inv_std = jax.lax.rsqrt(var + EPS)
    scale = inv_std * w_ref[...]
    scale_ref[...] = scale
    shift_ref[...] = b_ref[...] - m * scale


def _norm_kernel(ids_ref, x_ref, scale_ref, shift_ref, o_ref, *, rows):
    ids = ids_ref[...]
    iota = jax.lax.broadcasted_iota(jnp.int32, (rows, NB), 1)
    onehot = (ids == iota).astype(jnp.float32)
    dims = (((1,), (0,)), ((), ()))
    sc = jax.lax.dot_general(onehot, scale_ref[...], dims,
                             preferred_element_type=jnp.float32,
                             precision=_PREC)
    sh = jax.lax.dot_general(onehot, shift_ref[...], dims,
                             preferred_element_type=jnp.float32,
                             precision=_PREC)
    o_ref[...] = x_ref[...] * sc + sh


def kernel(data, batch_id, weights, bias):
    n = data.shape[0]
    rows = _pick_rows(n)
    nblk = n // rows
    ids = batch_id.astype(jnp.int32).reshape(n, 1)

    s1, s2, cnt = pl.pallas_call(
        functools.partial(_stats_kernel, rows=rows),
        grid=(nblk,),
        in_specs=[
            pl.BlockSpec((rows, 1), lambda i: (i, 0)),
            pl.BlockSpec((rows, IC), lambda i: (i, 0)),
        ],
        out_specs=[
            pl.BlockSpec((NB, IC), lambda i: (0, 0)),
            pl.BlockSpec((NB, IC), lambda i: (0, 0)),
            pl.BlockSpec((NB, 128), lambda i: (0, 0)),
        ],
        out_shape=[
            jax.ShapeDtypeStruct((NB, IC), jnp.float32),
            jax.ShapeDtypeStruct((NB, IC), jnp.float32),
            jax.ShapeDtypeStruct((NB, 128), jnp.float32),
        ],
    )(ids, data)

    gmat = jnp.asarray(np.kron(np.eye(NGROUPS, dtype=np.float32),
                               np.ones((CPG, CPG), np.float32)))
    scale, shift = pl.pallas_call(
        _table_kernel,
        out_shape=[
            jax.ShapeDtypeStruct((NB, IC), jnp.float32),
            jax.ShapeDtypeStruct((NB, IC), jnp.float32),
        ],
    )(s1, s2, cnt, gmat, weights, bias)

    out = pl.pallas_call(
        functools.partial(_norm_kernel, rows=rows),
        grid=(nblk,),
        in_specs=[
            pl.BlockSpec((rows, 1), lambda i: (i, 0)),
            pl.BlockSpec((rows, IC), lambda i: (i, 0)),
            pl.BlockSpec((NB, IC), lambda i: (0, 0)),
            pl.BlockSpec((NB, IC), lambda i: (0, 0)),
        ],
        out_specs=pl.BlockSpec((rows, IC), lambda i: (i, 0)),
        out_shape=jax.ShapeDtypeStruct((n, IC), jnp.float32),
    )(ids, data, scale, shift)
    return out


# speedup vs baseline: 5.4794x; 1.2476x over previous
import functools

import jax
import jax.numpy as jnp
import numpy as np
from jax.experimental import pallas as pl
from jax.experimental.pallas import tpu as pltpu

IC = 512
NGROUPS = 32
CPG = IC // NGROUPS
EPS = 1e-5
NB = 16

_PREC = jax.lax.Precision.HIGHEST


def _pick_rows(n):
    for r in (4000, 2000, 1000, 800, 400, 200, 80, 40, 8):
        if n % r == 0:
            return r
    return n


def _stats_kernel(ids_ref, x_ref, s1_ref, s2_ref, cnt_ref, *, rows):
    x = x_ref[...]
    ids = ids_ref[...]
    iota = jax.lax.broadcasted_iota(jnp.int32, (rows, NB), 1)
    onehot = (ids == iota).astype(jnp.bfloat16)
    dims = (((0,), (0,)), ((), ()))
    s1 = jax.lax.dot_general(onehot, x.astype(jnp.bfloat16), dims,
                             preferred_element_type=jnp.float32)
    s2 = jax.lax.dot_general(onehot, (x * x).astype(jnp.bfloat16), dims,
                             preferred_element_type=jnp.float32)
    ones = jnp.ones((rows, 128), jnp.bfloat16)
    cnt = jax.lax.dot_general(onehot, ones, dims,
                              preferred_element_type=jnp.float32)

    @pl.when(pl.program_id(0) == 0)
    def _init():
        s1_ref[...] = s1
        s2_ref[...] = s2
        cnt_ref[...] = cnt

    @pl.when(pl.program_id(0) != 0)
    def _acc():
        s1_ref[...] += s1
        s2_ref[...] += s2
        cnt_ref[...] += cnt


def _table_kernel(s1_ref, s2_ref, cnt_ref, g_ref, w_ref, b_ref,
                  scale_ref, shift_ref):
    n16 = cnt_ref[...][:, 0:1] * CPG
    ic = 1.0 / (n16 + EPS)
    dims = (((1,), (0,)), ((), ()))
    s1g = jax.lax.dot_general(s1_ref[...], g_ref[...], dims,
                              preferred_element_type=jnp.float32,
                              precision=_PREC)
    s2g = jax.lax.dot_general(s2_ref[...], g_ref[...], dims,
                              preferred_element_type=jnp.float32,
                              precision=_PREC)
    m = s1g * ic
    q = n16 * ic
    var = s2g * ic - m * m * (2.0 - q)
    inv_std = jax.lax.rsqrt(var + EPS)
    scale = inv_std * w_ref[...]
    scale_ref[...] = scale
    shift_ref[...] = b_ref[...] - m * scale


def _norm_kernel(ids_ref, x_ref, scale_ref, shift_ref, o_ref, *, rows):
    ids = ids_ref[...]
    iota = jax.lax.broadcasted_iota(jnp.int32, (rows, NB), 1)
    onehot = (ids == iota).astype(jnp.float32)
    dims = (((1,), (0,)), ((), ()))
    sc = jax.lax.dot_general(onehot, scale_ref[...], dims,
                             preferred_element_type=jnp.float32,
                             precision=_PREC)
    sh = jax.lax.dot_general(onehot, shift_ref[...], dims,
                             preferred_element_type=jnp.float32,
                             precision=_PREC)
    o_ref[...] = x_ref[...] * sc + sh


def kernel(data, batch_id, weights, bias):
    n = data.shape[0]
    rows = _pick_rows(n)
    nblk = n // rows
    ids = batch_id.astype(jnp.int32).reshape(n, 1)

    s1, s2, cnt = pl.pallas_call(
        functools.partial(_stats_kernel, rows=rows),
        grid=(nblk,),
        in_specs=[
            pl.BlockSpec((rows, 1), lambda i: (i, 0)),
            pl.BlockSpec((rows, IC), lambda i: (i, 0)),
        ],
        out_specs=[
            pl.BlockSpec((NB, IC), lambda i: (0, 0)),
            pl.BlockSpec((NB, IC), lambda i: (0, 0)),
            pl.BlockSpec((NB, 128), lambda i: (0, 0)),
        ],
        out_shape=[
            jax.ShapeDtypeStruct((NB, IC), jnp.float32),
            jax.ShapeDtypeStruct((NB, IC), jnp.float32),
            jax.ShapeDtypeStruct((NB, 128), jnp.float32),
        ],
    )(ids, data)

    gmat = jnp.asarray(np.kron(np.eye(NGROUPS, dtype=np.float32),
                               np.ones((CPG, CPG), np.float32)))
    scale, shift = pl.pallas_call(
        _table_kernel,
        out_shape=[
            jax.ShapeDtypeStruct((NB, IC), jnp.float32),
            jax.ShapeDtypeStruct((NB, IC), jnp.float32),
        ],
    )(s1, s2, cnt, gmat, weights, bias)

    out = pl.pallas_call(
        functools.partial(_norm_kernel, rows=rows),
        grid=(nblk,),
        in_specs=[
            pl.BlockSpec((rows, 1), lambda i: (i, 0)),
            pl.BlockSpec((rows, IC), lambda i: (i, 0)),
            pl.BlockSpec((NB, IC), lambda i: (0, 0)),
            pl.BlockSpec((NB, IC), lambda i: (0, 0)),
        ],
        out_specs=pl.BlockSpec((rows, IC), lambda i: (i, 0)),
        out_shape=jax.ShapeDtypeStruct((n, IC), jnp.float32),
    )(ids, data, scale, shift)
    return out


# speedup vs baseline: 11.8872x; 2.1694x over previous
import functools

import jax
import jax.numpy as jnp
import numpy as np
from jax.experimental import pallas as pl
from jax.experimental.pallas import tpu as pltpu

IC = 512
NGROUPS = 32
CPG = IC // NGROUPS
EPS = 1e-5
NB = 16

_PREC = jax.lax.Precision.HIGHEST


def _pick_rows(n):
    for r in (4000, 2000, 1000, 800, 400, 200, 80, 40, 8):
        if n % r == 0:
            return r
    return n


def _onehot_t(ids_ref, rows):
    ids = ids_ref[...].reshape(1, rows)
    biota = jax.lax.broadcasted_iota(jnp.int32, (NB, rows), 0)
    return (ids == biota).astype(jnp.bfloat16)


def _stats_kernel(ids_ref, x_ref, s1_ref, s2_ref, cnt_ref, *, rows):
    x = x_ref[...]
    oht = _onehot_t(ids_ref, rows)
    dims = (((1,), (0,)), ((), ()))
    s1 = jax.lax.dot_general(oht, x.astype(jnp.bfloat16), dims,
                             preferred_element_type=jnp.float32)
    s2 = jax.lax.dot_general(oht, (x * x).astype(jnp.bfloat16), dims,
                             preferred_element_type=jnp.float32)
    ones = jnp.ones((rows, 128), jnp.bfloat16)
    cnt = jax.lax.dot_general(oht, ones, dims,
                              preferred_element_type=jnp.float32)

    @pl.when(pl.program_id(0) == 0)
    def _init():
        s1_ref[...] = s1
        s2_ref[...] = s2
        cnt_ref[...] = cnt

    @pl.when(pl.program_id(0) != 0)
    def _acc():
        s1_ref[...] += s1
        s2_ref[...] += s2
        cnt_ref[...] += cnt


def _table_kernel(s1_ref, s2_ref, cnt_ref, g_ref, w_ref, b_ref,
                  scale_ref, shift_ref):
    n16 = cnt_ref[...][:, 0:1] * CPG
    ic = 1.0 / (n16 + EPS)
    dims = (((1,), (0,)), ((), ()))
    s1g = jax.lax.dot_general(s1_ref[...], g_ref[...], dims,
                              preferred_element_type=jnp.float32,
                              precision=_PREC)
    s2g = jax.lax.dot_general(s2_ref[...], g_ref[...], dims,
                              preferred_element_type=jnp.float32,
                              precision=_PREC)
    m = s1g * ic
    q = n16 * ic
    var = s2g * ic - m * m * (2.0 - q)
    inv_std = jax.lax.rsqrt(var + EPS)
    scale = inv_std * w_ref[...]
    scale_ref[...] = scale
    shift_ref[...] = b_ref[...] - m * scale


def _norm_kernel(ids_ref, x_ref, scale_ref, shift_ref, o_ref, *, rows):
    oht = _onehot_t(ids_ref, rows)
    dims = (((0,), (0,)), ((), ()))

    def expand(tbl):
        hi = tbl.astype(jnp.bfloat16)
        lo = (tbl - hi.astype(jnp.float32)).astype(jnp.bfloat16)
        out = jax.lax.dot_general(oht, hi, dims,
                                  preferred_element_type=jnp.float32)
        return out + jax.lax.dot_general(oht, lo, dims,
                                         preferred_element_type=jnp.float32)

    sc = expand(scale_ref[...])
    sh = expand(shift_ref[...])
    o_ref[...] = x_ref[...] * sc + sh


def kernel(data, batch_id, weights, bias):
    n = data.shape[0]
    rows = _pick_rows(n)
    nblk = n // rows
    ids = batch_id.astype(jnp.int32).reshape(nblk, 1, rows)

    s1, s2, cnt = pl.pallas_call(
        functools.partial(_stats_kernel, rows=rows),
        grid=(nblk,),
        in_specs=[
            pl.BlockSpec((1, 1, rows), lambda i: (i, 0, 0)),
            pl.BlockSpec((rows, IC), lambda i: (i, 0)),
        ],
        out_specs=[
            pl.BlockSpec((NB, IC), lambda i: (0, 0)),
            pl.BlockSpec((NB, IC), lambda i: (0, 0)),
            pl.BlockSpec((NB, 128), lambda i: (0, 0)),
        ],
        out_shape=[
            jax.ShapeDtypeStruct((NB, IC), jnp.float32),
            jax.ShapeDtypeStruct((NB, IC), jnp.float32),
            jax.ShapeDtypeStruct((NB, 128), jnp.float32),
        ],
    )(ids, data)

    gmat = jnp.asarray(np.kron(np.eye(NGROUPS, dtype=np.float32),
                               np.ones((CPG, CPG), np.float32)))
    scale, shift = pl.pallas_call(
        _table_kernel,
        out_shape=[
            jax.ShapeDtypeStruct((NB, IC), jnp.float32),
            jax.ShapeDtypeStruct((NB, IC), jnp.float32),
        ],
    )(s1, s2, cnt, gmat, weights, bias)

    out = pl.pallas_call(
        functools.partial(_norm_kernel, rows=rows),
        grid=(nblk,),
        in_specs=[
            pl.BlockSpec((1, 1, rows), lambda i: (i, 0, 0)),
            pl.BlockSpec((rows, IC), lambda i: (i, 0)),
            pl.BlockSpec((NB, IC), lambda i: (0, 0)),
            pl.BlockSpec((NB, IC), lambda i: (0, 0)),
        ],
        out_specs=pl.BlockSpec((rows, IC), lambda i: (i, 0)),
        out_shape=jax.ShapeDtypeStruct((n, IC), jnp.float32),
    )(ids, data, scale, shift)
    return out
